# baseline (speedup 1.0000x reference)
import jax
import jax.numpy as jnp
from jax.experimental import pallas as pl
from jax.experimental.pallas import tpu as pltpu


def _gcn_block(x_ref, adj_ref, wt_ref, b_ref, out_ref, xbf_ref):
    @pl.when(pl.program_id(0) == 0)
    def _cache_x():
        xbf_ref[...] = x_ref[...].astype(jnp.bfloat16)

    adj_bf = adj_ref[...].astype(jnp.bfloat16)
    h = jnp.dot(adj_bf, xbf_ref[...], preferred_element_type=jnp.float32)
    out_ref[...] = (
        jnp.dot(h, wt_ref[...], preferred_element_type=jnp.float32) + b_ref[...]
    )


def kernel(x, adj, W, b):
    n, d_in = x.shape
    d_out = W.shape[0]
    bm = 512
    wt = W.T
    b2 = b.reshape(1, d_out)
    return pl.pallas_call(
        _gcn_block,
        grid=(pl.cdiv(n, bm),),
        in_specs=[
            pl.BlockSpec((n, d_in), lambda i: (0, 0)),
            pl.BlockSpec((bm, n), lambda i: (i, 0)),
            pl.BlockSpec((d_in, d_out), lambda i: (0, 0)),
            pl.BlockSpec((1, d_out), lambda i: (0, 0)),
        ],
        out_specs=pl.BlockSpec((bm, d_out), lambda i: (i, 0)),
        out_shape=jax.ShapeDtypeStruct((n, d_out), jnp.float32),
        scratch_shapes=[pltpu.VMEM((n, d_in), jnp.bfloat16)],
        compiler_params=pltpu.CompilerParams(
            dimension_semantics=("arbitrary",),
        ),
    )(x, adj, wt, b2)
